# baseline (speedup 1.0000x reference)
import functools

import jax
import jax.numpy as jnp
from jax import lax
from jax.experimental import pallas as pl
from jax.experimental.pallas import tpu as pltpu
from jax.experimental.pallas import tpu_sc as plsc

N = 10000
D = 128
E = 320000
K_HOPS = 4
EPS = 1e-5

NC = 2
NS = 16
DH = D // NC
NPAD = 10240
RPT = NPAD // NS
CH = 128
EPW = 20736
NCHUNK = EPW // CH
NPAIR = NCHUNK // 6
EPAD = EPW * NS
ERW = EPW // 128


def _tc_pre_body(x_ref, g_ref, bt_ref, w_ref, b_ref, o_ref):
    x = x_ref[...]
    mean = jnp.mean(x, axis=0, keepdims=True)
    var = jnp.mean(x * x, axis=0, keepdims=True) - mean * mean
    h = (x - mean) * lax.rsqrt(var + EPS) * g_ref[...] + bt_ref[...]
    h = jnp.tanh(
        lax.dot_general(h, w_ref[...], (((1,), (1,)), ((), ())),
                        preferred_element_type=jnp.float32) + b_ref[...])
    o_ref[0, :N] = h[:, :DH]
    o_ref[1, :N] = h[:, DH:]
    z = jnp.zeros((NPAD - N, DH), jnp.float32)
    o_ref[0, N:] = z
    o_ref[1, N:] = z


def _tc_post_body(p_ref, wsg_ref, bsg_ref, wout_ref, bout_ref, o_ref):
    h = jnp.concatenate([p_ref[0, :N], p_ref[1, :N]], axis=1)
    h = jnp.tanh(
        lax.dot_general(h, wsg_ref[...], (((1,), (1,)), ((), ())),
                        preferred_element_type=jnp.float32) + bsg_ref[...])
    o_ref[...] = lax.dot_general(
        h, wout_ref[...], (((1,), (1,)), ((), ())),
        preferred_element_type=jnp.float32) + bout_ref[...]


def _spmm_body(h_hbm, src_hbm, dst_hbm, w_hbm, out_hbm,
               acc, src_v, dst_v, wst0, wst1, wst2,
               gbuf0, gbuf1, gbuf2, sbuf0, sbuf1,
               sem_g0, sem_g1, sem_g2, sem_s0, sem_s1,
               sem_w0, sem_w1, sem_w2):
    cid = lax.axis_index("c")
    sid = lax.axis_index("s")
    gbufs = (gbuf0, gbuf1, gbuf2)
    sbufs = (sbuf0, sbuf1)
    wsts = (wst0, wst1, wst2)
    sem_g = (sem_g0, sem_g1, sem_g2)
    sem_s = (sem_s0, sem_s1)
    sem_w = (sem_w0, sem_w1, sem_w2)
    hpl = h_hbm.at[cid]

    pltpu.sync_copy(src_hbm.at[pl.ds(sid * ERW, ERW)], src_v)
    pltpu.sync_copy(dst_hbm.at[pl.ds(sid * ERW, ERW)], dst_v)

    def zrow(i, carry):
        for k in range(DH // 16):
            sbuf0[i, pl.ds(k * 16, 16)] = jnp.zeros((16,), jnp.float32)
        return carry
    lax.fori_loop(0, CH, zrow, 0)
    for r in range(RPT // CH):
        pltpu.sync_copy(sbuf0, acc.at[pl.ds(sid * RPT + r * CH, CH)])
    plsc.subcore_barrier()

    for b in range(3):
        pltpu.async_copy(w_hbm.at[pl.ds(sid * EPW + b * CH, CH)],
                         wsts[b], sem_w[b])
        pltpu.async_copy(hpl.at[src_v.at[b]], gbufs[b], sem_g[b])

    def pair_body(q, carry):
        for i in range(6):
            g = q * 6 + i
            b = i % 3
            c = i % 2
            pltpu.make_async_copy(hpl.at[src_v.at[g]],
                                  gbufs[b], sem_g[b]).wait()
            pltpu.make_async_copy(w_hbm.at[pl.ds(sid * EPW + g * CH, CH)],
                                  wsts[b], sem_w[b]).wait()

            @pl.when((q > 0) | (i >= 2))
            def _():
                pltpu.make_async_copy(sbufs[c], acc.at[dst_v.at[g]],
                                      sem_s[c]).wait()

            def group(u, carry2):
                e0 = u * 16
                w16 = wsts[b][pl.ds(e0, 16)]
                for j in range(16):
                    wb = lax.gather(
                        w16, jnp.full((16, 1), j, jnp.int32),
                        dimension_numbers=lax.GatherDimensionNumbers(
                            offset_dims=(), collapsed_slice_dims=(0,),
                            start_index_map=(0,)),
                        slice_sizes=(1,),
                        mode=lax.GatherScatterMode.PROMISE_IN_BOUNDS)
                    for k in range(DH // 16):
                        sbufs[c][e0 + j, pl.ds(k * 16, 16)] = (
                            gbufs[b][e0 + j, pl.ds(k * 16, 16)] * wb)
                return carry2
            lax.fori_loop(0, CH // 16, group, 0)

            pltpu.async_copy(sbufs[c], acc.at[dst_v.at[g]], sem_s[c],
                             add=True)

            @pl.when(g + 3 < NCHUNK)
            def _():
                pltpu.async_copy(hpl.at[src_v.at[g + 3]], gbufs[b], sem_g[b])
                pltpu.async_copy(
                    w_hbm.at[pl.ds(sid * EPW + (g + 3) * CH, CH)],
                    wsts[b], sem_w[b])
        return carry
    lax.fori_loop(0, NPAIR, pair_body, 0)
    for c in range(2):
        g = NCHUNK - 2 + c
        pltpu.make_async_copy(sbufs[c], acc.at[dst_v.at[g]],
                              sem_s[c]).wait()

    plsc.subcore_barrier()
    pltpu.sync_copy(acc.at[pl.ds(sid * RPT, RPT)],
                    out_hbm.at[cid, pl.ds(sid * RPT, RPT)])


_spmm_kernel = functools.partial(
    pl.kernel,
    out_type=jax.ShapeDtypeStruct((NC, NPAD, DH), jnp.float32),
    mesh=plsc.VectorSubcoreMesh(core_axis_name="c", subcore_axis_name="s"),
    compiler_params=pltpu.CompilerParams(needs_layout_passes=False, use_tc_tiling_on_sc=False),
    scratch_types=[
        pltpu.VMEM_SHARED((NPAD, DH), jnp.float32),
        pltpu.VMEM((ERW, 128), jnp.int32),
        pltpu.VMEM((ERW, 128), jnp.int32),
        pltpu.VMEM((CH,), jnp.float32),
        pltpu.VMEM((CH,), jnp.float32),
        pltpu.VMEM((CH,), jnp.float32),
        pltpu.VMEM((CH, DH), jnp.float32),
        pltpu.VMEM((CH, DH), jnp.float32),
        pltpu.VMEM((CH, DH), jnp.float32),
        pltpu.VMEM((CH, DH), jnp.float32),
        pltpu.VMEM((CH, DH), jnp.float32),
    ] + [pltpu.SemaphoreType.DMA] * 8,
)(_spmm_body)


def _tc_pre(x, g, bt, w, b):
    return pl.pallas_call(
        _tc_pre_body,
        out_shape=jax.ShapeDtypeStruct((NC, NPAD, DH), jnp.float32),
    )(x, g, bt, w, b)


def _tc_post(parts, wsg, bsg, wout, bout):
    return pl.pallas_call(
        _tc_post_body,
        out_shape=jax.ShapeDtypeStruct((N, D), jnp.float32),
    )(parts, wsg, bsg, wout, bout)


def kernel(x, edge_index, edge_weight, bn_gamma, bn_beta,
           W_in, b_in, W_sg, b_sg, W_out, b_out):
    pad = EPAD - E
    dst = jnp.concatenate([edge_index[0], jnp.zeros((pad,), jnp.int32)])
    src = jnp.concatenate([edge_index[1], jnp.zeros((pad,), jnp.int32)])
    w2 = jnp.concatenate([edge_weight, jnp.zeros((pad,), jnp.float32)])
    dst2 = dst.reshape(EPAD // 128, 128)
    src2 = src.reshape(EPAD // 128, 128)

    g = bn_gamma.reshape(1, D)
    bt = bn_beta.reshape(1, D)
    b = b_in.reshape(1, D)
    bsg = b_sg.reshape(1, D)
    bout = b_out.reshape(1, D)

    h = _tc_pre(x, g, bt, W_in, b)
    for _ in range(K_HOPS):
        h = _spmm_kernel(h, src2, dst2, w2)
    return _tc_post(h, W_sg, bsg, W_out, bout)


# speedup vs baseline: 1.2048x; 1.2048x over previous
import functools

import jax
import jax.numpy as jnp
from jax import lax
from jax.experimental import pallas as pl
from jax.experimental.pallas import tpu as pltpu
from jax.experimental.pallas import tpu_sc as plsc

N = 10000
D = 128
E = 320000
K_HOPS = 4
EPS = 1e-5

NC = 2
NS = 16
DH = D // NC
NPAD = 10240
RPT = NPAD // NS
CH = 128
EPW = 20480
NCHUNK = EPW // CH
NPAIR = NCHUNK // 2
EPAD = EPW * NS
ERW = EPW // 128


def _tc_pre_body(x_ref, g_ref, bt_ref, w_ref, b_ref, o_ref):
    x = x_ref[...]
    mean = jnp.mean(x, axis=0, keepdims=True)
    var = jnp.mean(x * x, axis=0, keepdims=True) - mean * mean
    h = (x - mean) * lax.rsqrt(var + EPS) * g_ref[...] + bt_ref[...]
    h = jnp.tanh(
        lax.dot_general(h, w_ref[...], (((1,), (1,)), ((), ())),
                        preferred_element_type=jnp.float32) + b_ref[...])
    o_ref[0, :N] = h[:, :DH]
    o_ref[1, :N] = h[:, DH:]
    z = jnp.zeros((NPAD - N, DH), jnp.float32)
    o_ref[0, N:] = z
    o_ref[1, N:] = z


def _tc_post_body(p_ref, wsg_ref, bsg_ref, wout_ref, bout_ref, o_ref):
    h = jnp.concatenate([p_ref[0, :N], p_ref[1, :N]], axis=1)
    h = jnp.tanh(
        lax.dot_general(h, wsg_ref[...], (((1,), (1,)), ((), ())),
                        preferred_element_type=jnp.float32) + bsg_ref[...])
    o_ref[...] = lax.dot_general(
        h, wout_ref[...], (((1,), (1,)), ((), ())),
        preferred_element_type=jnp.float32) + bout_ref[...]


def _spmm_body(h_hbm, src_hbm, dst_hbm, w_hbm, out_hbm, tmp_hbm,
               acc, src_v, dst_v, wst0, wst1, gbuf0, gbuf1, sbuf0, sbuf1,
               sem_g0, sem_g1, sem_s0, sem_s1, sem_w0, sem_w1):
    cid = lax.axis_index("c")
    sid = lax.axis_index("s")
    gbufs = (gbuf0, gbuf1)
    sbufs = (sbuf0, sbuf1)
    wsts = (wst0, wst1)
    sem_g = (sem_g0, sem_g1)
    sem_s = (sem_s0, sem_s1)
    sem_w = (sem_w0, sem_w1)

    pltpu.sync_copy(src_hbm.at[pl.ds(sid * ERW, ERW)], src_v)
    pltpu.sync_copy(dst_hbm.at[pl.ds(sid * ERW, ERW)], dst_v)

    def run_hop(in_pl, out_pl):
        hpl = in_pl.at[cid]

        def zrow(i, carry):
            for k in range(DH // 16):
                sbuf0[i, pl.ds(k * 16, 16)] = jnp.zeros((16,), jnp.float32)
            return carry
        lax.fori_loop(0, CH, zrow, 0)
        for r in range(RPT // CH):
            pltpu.sync_copy(sbuf0, acc.at[pl.ds(sid * RPT + r * CH, CH)])
        plsc.subcore_barrier()

        for b in range(2):
            pltpu.async_copy(w_hbm.at[pl.ds(sid * EPW + b * CH, CH)],
                             wsts[b], sem_w[b])
            pltpu.async_copy(hpl.at[src_v.at[b]], gbufs[b], sem_g[b])

        def pair_body(q, carry):
            for b in range(2):
                g = q * 2 + b
                pltpu.make_async_copy(hpl.at[src_v.at[g]],
                                      gbufs[b], sem_g[b]).wait()
                pltpu.make_async_copy(
                    w_hbm.at[pl.ds(sid * EPW + g * CH, CH)],
                    wsts[b], sem_w[b]).wait()

                @pl.when(q > 0)
                def _():
                    pltpu.make_async_copy(sbufs[b], acc.at[dst_v.at[g]],
                                          sem_s[b]).wait()

                def group(u, carry2):
                    e0 = u * 16
                    w16 = wsts[b][pl.ds(e0, 16)]
                    for j in range(16):
                        wb = lax.gather(
                            w16, jnp.full((16, 1), j, jnp.int32),
                            dimension_numbers=lax.GatherDimensionNumbers(
                                offset_dims=(), collapsed_slice_dims=(0,),
                                start_index_map=(0,)),
                            slice_sizes=(1,),
                            mode=lax.GatherScatterMode.PROMISE_IN_BOUNDS)
                        for k in range(DH // 16):
                            sbufs[b][e0 + j, pl.ds(k * 16, 16)] = (
                                gbufs[b][e0 + j, pl.ds(k * 16, 16)] * wb)
                    return carry2
                lax.fori_loop(0, CH // 16, group, 0)

                pltpu.async_copy(sbufs[b], acc.at[dst_v.at[g]], sem_s[b],
                                 add=True)

                @pl.when(q < NPAIR - 1)
                def _():
                    pltpu.async_copy(hpl.at[src_v.at[g + 2]],
                                     gbufs[b], sem_g[b])
                    pltpu.async_copy(
                        w_hbm.at[pl.ds(sid * EPW + (g + 2) * CH, CH)],
                        wsts[b], sem_w[b])
            return carry
        lax.fori_loop(0, NPAIR, pair_body, 0)
        for b in range(2):
            g = NCHUNK - 2 + b
            pltpu.make_async_copy(sbufs[b], acc.at[dst_v.at[g]],
                                  sem_s[b]).wait()

        plsc.subcore_barrier()
        pltpu.sync_copy(acc.at[pl.ds(sid * RPT, RPT)],
                        out_pl.at[cid, pl.ds(sid * RPT, RPT)])
        plsc.subcore_barrier()

    run_hop(h_hbm, tmp_hbm)
    run_hop(tmp_hbm, out_hbm)
    run_hop(out_hbm, tmp_hbm)
    run_hop(tmp_hbm, out_hbm)


_spmm_kernel = functools.partial(
    pl.kernel,
    out_type=jax.ShapeDtypeStruct((NC, NPAD, DH), jnp.float32),
    mesh=plsc.VectorSubcoreMesh(core_axis_name="c", subcore_axis_name="s"),
    compiler_params=pltpu.CompilerParams(needs_layout_passes=False,
                                         use_tc_tiling_on_sc=False),
    scratch_types=[
        pltpu.HBM((NC, NPAD, DH), jnp.float32),
        pltpu.VMEM_SHARED((NPAD, DH), jnp.float32),
        pltpu.VMEM((ERW, 128), jnp.int32),
        pltpu.VMEM((ERW, 128), jnp.int32),
        pltpu.VMEM((CH,), jnp.float32),
        pltpu.VMEM((CH,), jnp.float32),
        pltpu.VMEM((CH, DH), jnp.float32),
        pltpu.VMEM((CH, DH), jnp.float32),
        pltpu.VMEM((CH, DH), jnp.float32),
        pltpu.VMEM((CH, DH), jnp.float32),
    ] + [pltpu.SemaphoreType.DMA] * 6,
)(_spmm_body)


def _tc_pre(x, g, bt, w, b):
    return pl.pallas_call(
        _tc_pre_body,
        out_shape=jax.ShapeDtypeStruct((NC, NPAD, DH), jnp.float32),
    )(x, g, bt, w, b)


def _tc_post(parts, wsg, bsg, wout, bout):
    return pl.pallas_call(
        _tc_post_body,
        out_shape=jax.ShapeDtypeStruct((N, D), jnp.float32),
    )(parts, wsg, bsg, wout, bout)


def kernel(x, edge_index, edge_weight, bn_gamma, bn_beta,
           W_in, b_in, W_sg, b_sg, W_out, b_out):
    pad = EPAD - E
    dst = jnp.concatenate([edge_index[0], jnp.zeros((pad,), jnp.int32)])
    src = jnp.concatenate([edge_index[1], jnp.zeros((pad,), jnp.int32)])
    w2 = jnp.concatenate([edge_weight, jnp.zeros((pad,), jnp.float32)])
    dst2 = dst.reshape(EPAD // 128, 128)
    src2 = src.reshape(EPAD // 128, 128)

    g = bn_gamma.reshape(1, D)
    bt = bn_beta.reshape(1, D)
    b = b_in.reshape(1, D)
    bsg = b_sg.reshape(1, D)
    bout = b_out.reshape(1, D)

    h = _tc_pre(x, g, bt, W_in, b)
    h = _spmm_kernel(h, src2, dst2, w2)
    return _tc_post(h, W_sg, bsg, W_out, bout)


# speedup vs baseline: 1.2972x; 1.0767x over previous
import functools

import jax
import jax.numpy as jnp
from jax import lax
from jax.experimental import pallas as pl
from jax.experimental.pallas import tpu as pltpu
from jax.experimental.pallas import tpu_sc as plsc

N = 10000
D = 128
E = 320000
K_HOPS = 4
EPS = 1e-5

NC = 2
NS = 16
DH = D // NC
NPAD = 10240
RPT = NPAD // NS
CH = 256
CR = CH // 128
EPW = 20480
NCHUNK = EPW // CH
EPAD = EPW * NS
ERW = EPW // 128


def _tc_pre_body(x_ref, g_ref, bt_ref, w_ref, b_ref, o_ref):
    x = x_ref[...]
    mean = jnp.mean(x, axis=0, keepdims=True)
    var = jnp.mean(x * x, axis=0, keepdims=True) - mean * mean
    h = (x - mean) * lax.rsqrt(var + EPS) * g_ref[...] + bt_ref[...]
    h = jnp.tanh(
        lax.dot_general(h, w_ref[...], (((1,), (1,)), ((), ())),
                        preferred_element_type=jnp.float32) + b_ref[...])
    o_ref[0, :N] = h[:, :DH]
    o_ref[1, :N] = h[:, DH:]
    z = jnp.zeros((NPAD - N, DH), jnp.float32)
    o_ref[0, N:] = z
    o_ref[1, N:] = z


def _tc_post_body(p_ref, wsg_ref, bsg_ref, wout_ref, bout_ref, o_ref):
    h = jnp.concatenate([p_ref[0, :N], p_ref[1, :N]], axis=1)
    h = jnp.tanh(
        lax.dot_general(h, wsg_ref[...], (((1,), (1,)), ((), ())),
                        preferred_element_type=jnp.float32) + bsg_ref[...])
    o_ref[...] = lax.dot_general(
        h, wout_ref[...], (((1,), (1,)), ((), ())),
        preferred_element_type=jnp.float32) + bout_ref[...]


def _spmm_body(h_hbm, src_hbm, dst_hbm, w_hbm, out_hbm,
               acc, srcst, dstst, wst0, wst1, gbuf0, gbuf1, sbuf0, sbuf1,
               sem_g0, sem_g1, sem_s0, sem_s1, sem_w0, sem_w1,
               sem_is0, sem_is1, sem_is2, sem_is3,
               sem_id0, sem_id1, sem_id2, sem_id3):
    cid = lax.axis_index("c")
    sid = lax.axis_index("s")
    gbufs = (gbuf0, gbuf1)
    sbufs = (sbuf0, sbuf1)
    wsts = (wst0, wst1)
    sem_g = (sem_g0, sem_g1)
    sem_s = (sem_s0, sem_s1)
    sem_w = (sem_w0, sem_w1)
    sem_is = (sem_is0, sem_is1, sem_is2, sem_is3)
    sem_id = (sem_id0, sem_id1, sem_id2, sem_id3)
    hpl = h_hbm.at[cid]
    erow = sid * ERW

    def stage_src(g, r):
        pltpu.async_copy(src_hbm.at[pl.ds(erow + g * CR, CR)],
                         srcst.at[r], sem_is[r])

    def stage_dst(g, r):
        pltpu.async_copy(dst_hbm.at[pl.ds(erow + g * CR, CR)],
                         dstst.at[r], sem_id[r])

    def wait_src(r):
        pltpu.make_async_copy(src_hbm.at[pl.ds(erow, CR)],
                              srcst.at[r], sem_is[r]).wait()

    def wait_dst(r):
        pltpu.make_async_copy(dst_hbm.at[pl.ds(erow, CR)],
                              dstst.at[r], sem_id[r]).wait()

    def issue_gather(r, b):
        for j in range(CR):
            pltpu.async_copy(hpl.at[srcst.at[r, j]],
                             gbufs[b].at[pl.ds(j * 128, 128)], sem_g[b])

    def wait_gather(r, b):
        for j in range(CR):
            pltpu.make_async_copy(hpl.at[srcst.at[r, j]],
                                  gbufs[b].at[pl.ds(j * 128, 128)],
                                  sem_g[b]).wait()

    def issue_scatter(r, b):
        for j in range(CR):
            pltpu.async_copy(sbufs[b].at[pl.ds(j * 128, 128)],
                             acc.at[dstst.at[r, j]], sem_s[b], add=True)

    def wait_scatter(r, b):
        for j in range(CR):
            pltpu.make_async_copy(sbufs[b].at[pl.ds(j * 128, 128)],
                                  acc.at[dstst.at[r, j]], sem_s[b]).wait()

    def zrow(i, carry):
        for k in range(DH // 16):
            sbuf0[i, pl.ds(k * 16, 16)] = jnp.zeros((16,), jnp.float32)
        return carry
    lax.fori_loop(0, CH, zrow, 0)
    for r in range(RPT // CH):
        pltpu.sync_copy(sbuf0, acc.at[pl.ds(sid * RPT + r * CH, CH)])
    plsc.subcore_barrier()

    for r in range(4):
        stage_src(r, r)
    for r in range(2):
        stage_dst(r, r)
    for b in range(2):
        pltpu.async_copy(w_hbm.at[pl.ds(sid * EPW + b * CH, CH)],
                         wsts[b], sem_w[b])
        wait_src(b)
        issue_gather(b, b)

    def quad_body(q, carry):
        for i in range(4):
            g = q * 4 + i
            b = i % 2
            wait_gather(i, b)
            pltpu.make_async_copy(
                w_hbm.at[pl.ds(sid * EPW + g * CH, CH)],
                wsts[b], sem_w[b]).wait()

            if i >= 2:
                wait_scatter(i, b)

                @pl.when(g + 2 < NCHUNK)
                def _():
                    stage_dst(g + 2, i - 2)
            else:
                @pl.when(q > 0)
                def _():
                    wait_scatter(i, b)

                @pl.when(g + 2 < NCHUNK)
                def _():
                    stage_dst(g + 2, i + 2)

            @pl.when(g + 4 < NCHUNK)
            def _():
                stage_src(g + 4, i)

            def group(u, carry2):
                e0 = u * 16
                w16 = wsts[b][pl.ds(e0, 16)]
                for j in range(16):
                    wb = lax.gather(
                        w16, jnp.full((16, 1), j, jnp.int32),
                        dimension_numbers=lax.GatherDimensionNumbers(
                            offset_dims=(), collapsed_slice_dims=(0,),
                            start_index_map=(0,)),
                        slice_sizes=(1,),
                        mode=lax.GatherScatterMode.PROMISE_IN_BOUNDS)
                    for k in range(DH // 16):
                        sbufs[b][e0 + j, pl.ds(k * 16, 16)] = (
                            gbufs[b][e0 + j, pl.ds(k * 16, 16)] * wb)
                return carry2
            lax.fori_loop(0, CH // 16, group, 0)

            wait_dst(i)
            issue_scatter(i, b)

            @pl.when(g + 2 < NCHUNK)
            def _():
                wait_src((i + 2) % 4)
                issue_gather((i + 2) % 4, b)
                pltpu.async_copy(
                    w_hbm.at[pl.ds(sid * EPW + (g + 2) * CH, CH)],
                    wsts[b], sem_w[b])
        return carry
    lax.fori_loop(0, NCHUNK // 4, quad_body, 0)
    for b in range(2):
        wait_scatter(2 + b, b)

    plsc.subcore_barrier()
    pltpu.sync_copy(acc.at[pl.ds(sid * RPT, RPT)],
                    out_hbm.at[cid, pl.ds(sid * RPT, RPT)])


_spmm_kernel = functools.partial(
    pl.kernel,
    out_type=jax.ShapeDtypeStruct((NC, NPAD, DH), jnp.float32),
    mesh=plsc.VectorSubcoreMesh(core_axis_name="c", subcore_axis_name="s"),
    compiler_params=pltpu.CompilerParams(needs_layout_passes=False,
                                         use_tc_tiling_on_sc=False),
    scratch_types=[
        pltpu.VMEM_SHARED((NPAD, DH), jnp.float32),
        pltpu.VMEM((4, CR, 128), jnp.int32),
        pltpu.VMEM((4, CR, 128), jnp.int32),
        pltpu.VMEM((CH,), jnp.float32),
        pltpu.VMEM((CH,), jnp.float32),
        pltpu.VMEM((CH, DH), jnp.float32),
        pltpu.VMEM((CH, DH), jnp.float32),
        pltpu.VMEM((CH, DH), jnp.float32),
        pltpu.VMEM((CH, DH), jnp.float32),
    ] + [pltpu.SemaphoreType.DMA] * 14,
)(_spmm_body)


def _tc_pre(x, g, bt, w, b):
    return pl.pallas_call(
        _tc_pre_body,
        out_shape=jax.ShapeDtypeStruct((NC, NPAD, DH), jnp.float32),
    )(x, g, bt, w, b)


def _tc_post(parts, wsg, bsg, wout, bout):
    return pl.pallas_call(
        _tc_post_body,
        out_shape=jax.ShapeDtypeStruct((N, D), jnp.float32),
    )(parts, wsg, bsg, wout, bout)


def kernel(x, edge_index, edge_weight, bn_gamma, bn_beta,
           W_in, b_in, W_sg, b_sg, W_out, b_out):
    pad = EPAD - E
    dst = jnp.concatenate([edge_index[0], jnp.zeros((pad,), jnp.int32)])
    src = jnp.concatenate([edge_index[1], jnp.zeros((pad,), jnp.int32)])
    w2 = jnp.concatenate([edge_weight, jnp.zeros((pad,), jnp.float32)])
    dst2 = dst.reshape(EPAD // 128, 128)
    src2 = src.reshape(EPAD // 128, 128)

    g = bn_gamma.reshape(1, D)
    bt = bn_beta.reshape(1, D)
    b = b_in.reshape(1, D)
    bsg = b_sg.reshape(1, D)
    bout = b_out.reshape(1, D)

    h = _tc_pre(x, g, bt, W_in, b)
    for _ in range(K_HOPS):
        h = _spmm_kernel(h, src2, dst2, w2)
    return _tc_post(h, W_sg, bsg, W_out, bout)
